# baseline (speedup 1.0000x reference)
import functools

import jax
import jax.numpy as jnp
from jax import lax
from jax.experimental import pallas as pl
from jax.experimental.pallas import tpu as pltpu
from jax.experimental.pallas import tpu_sc as plsc

N = 10000
E = 320000
IN = 128
H = 8
D = 16
HD = H * D
OUT = 64
NEG_SLOPE = 0.2

NC = 2
NS = 16
NW = NC * NS
L = 16

CHUNK = 64
CPW = 160
E_PAD = NW * CPW * CHUNK
E_ALLOC = E_PAD + 2 * CHUNK
NCH = E_ALLOC // CHUNK
NPAD = 10240
RPT = NPAD // NS
BLK = 1280

def _dyn_gather(v, idx):
    dn = lax.GatherDimensionNumbers(
        offset_dims=(), collapsed_slice_dims=(0,), start_index_map=(0,))
    return lax.gather(v, idx[:, None], dn, (1,),
                      mode=lax.GatherScatterMode.PROMISE_IN_BOUNDS)


AW = HD + L


def _edge_kernel(src_hbm, dst_hbm, h_hbm, la_hbm, lb_hbm, acc_hbm, s_hbm,
                 psa, pda, psb, pdb,
                 slr0, dlr0, hrows0, wbuf0,
                 slr1, dlr1, hrows1, wbuf1,
                 acc_sh, s_sh, sem0, sem1):
    cid = lax.axis_index("c")
    sid = lax.axis_index("s")
    wid = cid * NS + sid

    zv = jnp.zeros((L,), jnp.float32)

    @pl.loop(0, CHUNK)
    def _(i):
        for cc in range(H):
            hrows0[i, pl.ds(cc * L, L)] = zv
        wbuf0[i, :] = zv

    row0 = sid * RPT
    for k in range(RPT // CHUNK):
        pltpu.sync_copy(hrows0, acc_sh.at[pl.ds(row0 + k * CHUNK, CHUNK)])
        pltpu.sync_copy(wbuf0, s_sh.at[pl.ds(row0 + k * CHUNK, CHUNK)])

    plsc.subcore_barrier()

    def fetch_pair(j, ps, pd):
        row = wid * CPW + j
        pltpu.sync_copy(src_hbm.at[pl.ds(row, 2)], ps)
        pltpu.sync_copy(dst_hbm.at[pl.ds(row, 2)], pd)

    def fire(si, di, hr, sl, dl, sem):
        pltpu.async_copy(h_hbm.at[si], hr, sem)
        pltpu.async_copy(la_hbm.at[si], sl, sem)
        pltpu.async_copy(lb_hbm.at[di], dl, sem)

    def drain(si, di, hr, sl, dl, sem):
        pltpu.make_async_copy(h_hbm.at[si], hr, sem).wait()
        pltpu.make_async_copy(la_hbm.at[si], sl, sem).wait()
        pltpu.make_async_copy(lb_hbm.at[di], dl, sem).wait()

    def compute(sl, dl, hr, wb, di):
        @plsc.parallel_loop(0, CHUNK, 1, unroll=4)
        def _(e):
            e16 = sl[e, :] + dl[e, :]
            lk = jnp.where(e16 > 0, e16, NEG_SLOPE * e16)
            w = jnp.exp(lk)
            wb[e, :] = w
            for hh in range(H):
                wh = _dyn_gather(w, jnp.full((L,), hh, jnp.int32))
                slc = pl.ds(hh * L, L)
                hr[e, slc] = hr[e, slc] * wh

        pltpu.sync_copy(wb, s_sh.at[di], add=True)
        pltpu.sync_copy(hr, acc_sh.at[di], add=True)

    fetch_pair(0, psa, pda)
    fire(psa.at[0], pda.at[0], hrows0, slr0, dlr0, sem0)
    fire(psa.at[1], pda.at[1], hrows1, slr1, dlr1, sem1)

    @pl.loop(0, CPW // 4)
    def _(u):
        j = u * 4
        fetch_pair(j + 2, psb, pdb)
        drain(psa.at[0], pda.at[0], hrows0, slr0, dlr0, sem0)
        compute(slr0, dlr0, hrows0, wbuf0, pda.at[0])
        fire(psb.at[0], pdb.at[0], hrows0, slr0, dlr0, sem0)
        drain(psa.at[1], pda.at[1], hrows1, slr1, dlr1, sem1)
        compute(slr1, dlr1, hrows1, wbuf1, pda.at[1])
        fire(psb.at[1], pdb.at[1], hrows1, slr1, dlr1, sem1)
        fetch_pair(j + 4, psa, pda)
        drain(psb.at[0], pdb.at[0], hrows0, slr0, dlr0, sem0)
        compute(slr0, dlr0, hrows0, wbuf0, pdb.at[0])
        fire(psa.at[0], pda.at[0], hrows0, slr0, dlr0, sem0)
        drain(psb.at[1], pdb.at[1], hrows1, slr1, dlr1, sem1)
        compute(slr1, dlr1, hrows1, wbuf1, pdb.at[1])
        fire(psa.at[1], pda.at[1], hrows1, slr1, dlr1, sem1)

    drain(psa.at[0], pda.at[0], hrows0, slr0, dlr0, sem0)
    drain(psa.at[1], pda.at[1], hrows1, slr1, dlr1, sem1)

    plsc.subcore_barrier()

    for k in range(RPT // CHUNK):
        r = row0 + k * CHUNK
        pltpu.sync_copy(acc_sh.at[pl.ds(r, CHUNK)], hrows0)
        pltpu.sync_copy(hrows0, acc_hbm.at[cid, pl.ds(r, CHUNK)])
        pltpu.sync_copy(s_sh.at[pl.ds(r, CHUNK)], wbuf0)
        pltpu.sync_copy(wbuf0, s_hbm.at[cid, pl.ds(r, CHUNK)])


@functools.cache
def _edge_call():
  mesh = plsc.VectorSubcoreMesh(
      core_axis_name="c", subcore_axis_name="s", num_cores=NC, num_subcores=NS)
  return pl.kernel(
    _edge_kernel,
    out_type=(
        jax.ShapeDtypeStruct((NC, NPAD, HD), jnp.float32),
        jax.ShapeDtypeStruct((NC, NPAD, L), jnp.float32),
    ),
    mesh=mesh,
    scratch_types=[
        pltpu.VMEM((2, CHUNK), jnp.int32),
        pltpu.VMEM((2, CHUNK), jnp.int32),
        pltpu.VMEM((2, CHUNK), jnp.int32),
        pltpu.VMEM((2, CHUNK), jnp.int32),
        pltpu.VMEM((CHUNK, L), jnp.float32),
        pltpu.VMEM((CHUNK, L), jnp.float32),
        pltpu.VMEM((CHUNK, HD), jnp.float32),
        pltpu.VMEM((CHUNK, L), jnp.float32),
        pltpu.VMEM((CHUNK, L), jnp.float32),
        pltpu.VMEM((CHUNK, L), jnp.float32),
        pltpu.VMEM((CHUNK, HD), jnp.float32),
        pltpu.VMEM((CHUNK, L), jnp.float32),
        pltpu.VMEM_SHARED((NPAD, HD), jnp.float32),
        pltpu.VMEM_SHARED((NPAD, L), jnp.float32),
        pltpu.SemaphoreType.DMA,
        pltpu.SemaphoreType.DMA,
    ],
    compiler_params=pltpu.CompilerParams(use_tc_tiling_on_sc=False),
  )


def _pre_kernel(x_ref, w_ref, alr_ref, h_ref, la_ref, lb_ref):
    i = pl.program_id(0)
    rows = i * BLK + lax.broadcasted_iota(jnp.int32, (BLK, 1), 0)
    valid = rows < N
    h = jnp.dot(x_ref[...], w_ref[...], preferred_element_type=jnp.float32)
    h = jnp.where(valid, h, 0.0)
    lr = jnp.dot(h, alr_ref[...], preferred_element_type=jnp.float32)
    lr = jnp.where(valid, lr, 0.0)
    h_ref[...] = h
    la_ref[...] = lr[:, :L]
    lb_ref[...] = lr[:, L:]


def _normalize(acc_ref, s_ref, p_ref, b_ref):
    t = acc_ref[0] + acc_ref[1]
    s = s_ref[0] + s_ref[1]
    s = jnp.where(s > 0, s, 1.0)
    srep = jnp.dot(s, p_ref[...], preferred_element_type=jnp.float32)
    return t / srep + b_ref[...]


def _mid_body(acc_ref, s_ref, p_ref, b_ref, gm_ref, bt_ref, w_ref, alr_ref,
              b2_ref, h_ref, la_ref, lb_ref):
    i = pl.program_id(0)
    rows = i * BLK + lax.broadcasted_iota(jnp.int32, (BLK, 1), 0)
    valid = rows < N
    y = _normalize(acc_ref, s_ref, p_ref, b_ref)
    mu = jnp.mean(y, axis=-1, keepdims=True)
    var = jnp.mean((y - mu) ** 2, axis=-1, keepdims=True)
    z = (y - mu) * lax.rsqrt(var + 1e-5) * gm_ref[...] + bt_ref[...]
    z = jnp.maximum(z, 0.0)
    h = jnp.dot(z, w_ref[...], preferred_element_type=jnp.float32) + b2_ref[...]
    h = jnp.where(valid, h, 0.0)
    lr = jnp.dot(h, alr_ref[...], preferred_element_type=jnp.float32)
    lr = jnp.where(valid, lr, 0.0)
    h_ref[...] = h
    la_ref[...] = lr[:, :L]
    lb_ref[...] = lr[:, L:]


def _row_spec(width):
    return pl.BlockSpec((BLK, width), lambda i: (i, 0))


def _full_spec(shape):
    nd = len(shape)
    return pl.BlockSpec(shape, lambda i, _n=nd: (0,) * _n)


def _acc_spec(width):
    return pl.BlockSpec((NC, BLK, width), lambda i: (0, i, 0))


_pre_call = pl.pallas_call(
    _pre_kernel,
    grid=(NPAD // BLK,),
    in_specs=[_row_spec(IN), _full_spec((IN, HD)), _full_spec((HD, 2 * L))],
    out_specs=(_row_spec(HD), _row_spec(L), _row_spec(L)),
    out_shape=(jax.ShapeDtypeStruct((NPAD, HD), jnp.float32),
               jax.ShapeDtypeStruct((NPAD, L), jnp.float32),
               jax.ShapeDtypeStruct((NPAD, L), jnp.float32)),
)

_mid_call = pl.pallas_call(
    _mid_body,
    grid=(NPAD // BLK,),
    in_specs=[_acc_spec(HD), _acc_spec(L), _full_spec((L, HD)),
              _full_spec((1, HD)), _full_spec((1, HD)), _full_spec((1, HD)),
              _full_spec((HD, HD)), _full_spec((HD, 2 * L)),
              _full_spec((1, HD))],
    out_specs=(_row_spec(HD), _row_spec(L), _row_spec(L)),
    out_shape=(jax.ShapeDtypeStruct((NPAD, HD), jnp.float32),
               jax.ShapeDtypeStruct((NPAD, L), jnp.float32),
               jax.ShapeDtypeStruct((NPAD, L), jnp.float32)),
)


def _block_alr(al, ar):
    eye = jnp.eye(H, dtype=jnp.float32)
    bl = (eye[:, None, :] * al[:, :, None]).reshape(HD, H)
    br = (eye[:, None, :] * ar[:, :, None]).reshape(HD, H)
    z = jnp.zeros((HD, H), jnp.float32)
    return jnp.concatenate([bl, z, br, z], axis=1)


@jax.jit
def kernel(g, feats, W0, al0, ar0, b0, gm0, bt0, W1, al1, ar1, b1, gm1, bt1,
           Wp, bp):
    src = jnp.concatenate(
        [g[0], jnp.full((E_ALLOC - E,), N, jnp.int32)]
    ).astype(jnp.int32).reshape(NCH, CHUNK)
    dst = jnp.concatenate(
        [g[1], jnp.full((E_ALLOC - E,), N, jnp.int32)]
    ).astype(jnp.int32).reshape(NCH, CHUNK)

    xpad = jnp.zeros((NPAD, IN), jnp.float32).at[:N].set(feats)

    p = jnp.zeros((L, HD), jnp.float32)
    eye = jnp.eye(H, dtype=jnp.float32)
    p = p.at[:H].set(jnp.repeat(eye, D, axis=1))

    alr0 = _block_alr(al0, ar0)
    alr1 = _block_alr(al1, ar1)

    wp_pad = jnp.zeros((HD, HD), jnp.float32).at[:, :OUT].set(Wp)
    bp_pad = jnp.zeros((HD,), jnp.float32).at[:OUT].set(bp)
    ws = jnp.stack([W1, wp_pad])
    alrs = jnp.stack([alr1, jnp.zeros((HD, 2 * L), jnp.float32)])
    bs = jnp.stack([b0[None, :], b1[None, :]])
    gms = jnp.stack([gm0[None, :], gm1[None, :]])
    bts = jnp.stack([bt0[None, :], bt1[None, :]])
    b2s = jnp.stack([jnp.zeros((1, HD), jnp.float32), bp_pad[None, :]])

    h0, la0, lb0 = _pre_call(xpad, W0, alr0)

    def body(carry, x):
        h, la, lb = carry
        w_i, alr_i, b_i, gm_i, bt_i, b2_i = x
        acc, s = _edge_call()(src, dst, h, la, lb)
        h2, la2, lb2 = _mid_call(acc, s, p, b_i, gm_i, bt_i, w_i, alr_i, b2_i)
        return (h2, la2, lb2), None

    (hf, _, _), _ = lax.scan(body, (h0, la0, lb0), (ws, alrs, bs, gms, bts, b2s))
    return hf[:N, :OUT]


# speedup vs baseline: 1.0680x; 1.0680x over previous
import functools

import jax
import jax.numpy as jnp
from jax import lax
from jax.experimental import pallas as pl
from jax.experimental.pallas import tpu as pltpu
from jax.experimental.pallas import tpu_sc as plsc

N = 10000
E = 320000
IN = 128
H = 8
D = 16
HD = H * D
OUT = 64
NEG_SLOPE = 0.2

NC = 2
NS = 16
NW = NC * NS
L = 16

CHUNK = 64
CPW = 158
E_PAD = NW * CPW * CHUNK
E_ALLOC = E_PAD + CHUNK
NPAD = 10240
RPT = NPAD // NS
BLK = 1280

def _dyn_gather(v, idx):
    dn = lax.GatherDimensionNumbers(
        offset_dims=(), collapsed_slice_dims=(0,), start_index_map=(0,))
    return lax.gather(v, idx[:, None], dn, (1,),
                      mode=lax.GatherScatterMode.PROMISE_IN_BOUNDS)


AW = HD + L


def _edge_kernel(src_hbm, dst_hbm, h_hbm, la_hbm, lb_hbm, acc_hbm, s_hbm,
                 sidx0, didx0, slr0, dlr0, hrows0, wbuf0,
                 sidx1, didx1, slr1, dlr1, hrows1, wbuf1,
                 acc_sh, s_sh, sem0, sem1):
    cid = lax.axis_index("c")
    sid = lax.axis_index("s")
    wid = cid * NS + sid

    zv = jnp.zeros((L,), jnp.float32)

    @pl.loop(0, CHUNK)
    def _(i):
        for cc in range(H):
            hrows0[i, pl.ds(cc * L, L)] = zv
        wbuf0[i, :] = zv

    row0 = sid * RPT
    for k in range(RPT // CHUNK):
        pltpu.sync_copy(hrows0, acc_sh.at[pl.ds(row0 + k * CHUNK, CHUNK)])
        pltpu.sync_copy(wbuf0, s_sh.at[pl.ds(row0 + k * CHUNK, CHUNK)])

    plsc.subcore_barrier()

    def fetch_idx(j, si, di):
        base = (wid * CPW + j) * CHUNK
        pltpu.sync_copy(src_hbm.at[pl.ds(base, CHUNK)], si)
        pltpu.sync_copy(dst_hbm.at[pl.ds(base, CHUNK)], di)

    def fire(si, di, hr, sl, dl, sem):
        pltpu.async_copy(h_hbm.at[si], hr, sem)
        pltpu.async_copy(la_hbm.at[si], sl, sem)
        pltpu.async_copy(lb_hbm.at[di], dl, sem)

    def drain(si, di, hr, sl, dl, sem):
        pltpu.make_async_copy(h_hbm.at[si], hr, sem).wait()
        pltpu.make_async_copy(la_hbm.at[si], sl, sem).wait()
        pltpu.make_async_copy(lb_hbm.at[di], dl, sem).wait()

    def compute(sl, dl, hr, wb, di):
        @plsc.parallel_loop(0, CHUNK, 1, unroll=4)
        def _(e):
            e16 = sl[e, :] + dl[e, :]
            lk = jnp.where(e16 > 0, e16, NEG_SLOPE * e16)
            w = jnp.exp(lk)
            wb[e, :] = w
            for hh in range(H):
                wh = _dyn_gather(w, jnp.full((L,), hh, jnp.int32))
                slc = pl.ds(hh * L, L)
                hr[e, slc] = hr[e, slc] * wh

        pltpu.sync_copy(wb, s_sh.at[di], add=True)
        pltpu.sync_copy(hr, acc_sh.at[di], add=True)

    fetch_idx(0, sidx0, didx0)
    fire(sidx0, didx0, hrows0, slr0, dlr0, sem0)

    @pl.loop(0, CPW // 2)
    def _(t):
        j = t * 2
        fetch_idx(j + 1, sidx1, didx1)
        fire(sidx1, didx1, hrows1, slr1, dlr1, sem1)
        drain(sidx0, didx0, hrows0, slr0, dlr0, sem0)
        compute(slr0, dlr0, hrows0, wbuf0, didx0)
        fetch_idx(j + 2, sidx0, didx0)
        fire(sidx0, didx0, hrows0, slr0, dlr0, sem0)
        drain(sidx1, didx1, hrows1, slr1, dlr1, sem1)
        compute(slr1, dlr1, hrows1, wbuf1, didx1)

    drain(sidx0, didx0, hrows0, slr0, dlr0, sem0)

    plsc.subcore_barrier()

    for k in range(RPT // CHUNK):
        r = row0 + k * CHUNK
        pltpu.sync_copy(acc_sh.at[pl.ds(r, CHUNK)], hrows0)
        pltpu.sync_copy(hrows0, acc_hbm.at[cid, pl.ds(r, CHUNK)])
        pltpu.sync_copy(s_sh.at[pl.ds(r, CHUNK)], wbuf0)
        pltpu.sync_copy(wbuf0, s_hbm.at[cid, pl.ds(r, CHUNK)])


@functools.cache
def _edge_call():
  mesh = plsc.VectorSubcoreMesh(
      core_axis_name="c", subcore_axis_name="s", num_cores=NC, num_subcores=NS)
  return pl.kernel(
    _edge_kernel,
    out_type=(
        jax.ShapeDtypeStruct((NC, NPAD, HD), jnp.float32),
        jax.ShapeDtypeStruct((NC, NPAD, L), jnp.float32),
    ),
    mesh=mesh,
    scratch_types=[
        pltpu.VMEM((CHUNK,), jnp.int32),
        pltpu.VMEM((CHUNK,), jnp.int32),
        pltpu.VMEM((CHUNK, L), jnp.float32),
        pltpu.VMEM((CHUNK, L), jnp.float32),
        pltpu.VMEM((CHUNK, HD), jnp.float32),
        pltpu.VMEM((CHUNK, L), jnp.float32),
        pltpu.VMEM((CHUNK,), jnp.int32),
        pltpu.VMEM((CHUNK,), jnp.int32),
        pltpu.VMEM((CHUNK, L), jnp.float32),
        pltpu.VMEM((CHUNK, L), jnp.float32),
        pltpu.VMEM((CHUNK, HD), jnp.float32),
        pltpu.VMEM((CHUNK, L), jnp.float32),
        pltpu.VMEM_SHARED((NPAD, HD), jnp.float32),
        pltpu.VMEM_SHARED((NPAD, L), jnp.float32),
        pltpu.SemaphoreType.DMA,
        pltpu.SemaphoreType.DMA,
    ],
    compiler_params=pltpu.CompilerParams(use_tc_tiling_on_sc=False),
  )


def _pre_kernel(x_ref, w_ref, alr_ref, h_ref, la_ref, lb_ref):
    i = pl.program_id(0)
    rows = i * BLK + lax.broadcasted_iota(jnp.int32, (BLK, 1), 0)
    valid = rows < N
    h = jnp.dot(x_ref[...], w_ref[...], preferred_element_type=jnp.float32)
    h = jnp.where(valid, h, 0.0)
    lr = jnp.dot(h, alr_ref[...], preferred_element_type=jnp.float32)
    lr = jnp.where(valid, lr, 0.0)
    h_ref[...] = h
    la_ref[...] = lr[:, :L]
    lb_ref[...] = lr[:, L:]


def _normalize(acc_ref, s_ref, p_ref, b_ref):
    t = acc_ref[0] + acc_ref[1]
    s = s_ref[0] + s_ref[1]
    s = jnp.where(s > 0, s, 1.0)
    srep = jnp.dot(s, p_ref[...], preferred_element_type=jnp.float32)
    return t / srep + b_ref[...]


def _mid_body(acc_ref, s_ref, p_ref, b_ref, gm_ref, bt_ref, w_ref, alr_ref,
              b2_ref, h_ref, la_ref, lb_ref):
    i = pl.program_id(0)
    rows = i * BLK + lax.broadcasted_iota(jnp.int32, (BLK, 1), 0)
    valid = rows < N
    y = _normalize(acc_ref, s_ref, p_ref, b_ref)
    mu = jnp.mean(y, axis=-1, keepdims=True)
    var = jnp.mean((y - mu) ** 2, axis=-1, keepdims=True)
    z = (y - mu) * lax.rsqrt(var + 1e-5) * gm_ref[...] + bt_ref[...]
    z = jnp.maximum(z, 0.0)
    h = jnp.dot(z, w_ref[...], preferred_element_type=jnp.float32) + b2_ref[...]
    h = jnp.where(valid, h, 0.0)
    lr = jnp.dot(h, alr_ref[...], preferred_element_type=jnp.float32)
    lr = jnp.where(valid, lr, 0.0)
    h_ref[...] = h
    la_ref[...] = lr[:, :L]
    lb_ref[...] = lr[:, L:]


def _row_spec(width):
    return pl.BlockSpec((BLK, width), lambda i: (i, 0))


def _full_spec(shape):
    nd = len(shape)
    return pl.BlockSpec(shape, lambda i, _n=nd: (0,) * _n)


def _acc_spec(width):
    return pl.BlockSpec((NC, BLK, width), lambda i: (0, i, 0))


_pre_call = pl.pallas_call(
    _pre_kernel,
    grid=(NPAD // BLK,),
    in_specs=[_row_spec(IN), _full_spec((IN, HD)), _full_spec((HD, 2 * L))],
    out_specs=(_row_spec(HD), _row_spec(L), _row_spec(L)),
    out_shape=(jax.ShapeDtypeStruct((NPAD, HD), jnp.float32),
               jax.ShapeDtypeStruct((NPAD, L), jnp.float32),
               jax.ShapeDtypeStruct((NPAD, L), jnp.float32)),
)

_mid_call = pl.pallas_call(
    _mid_body,
    grid=(NPAD // BLK,),
    in_specs=[_acc_spec(HD), _acc_spec(L), _full_spec((L, HD)),
              _full_spec((1, HD)), _full_spec((1, HD)), _full_spec((1, HD)),
              _full_spec((HD, HD)), _full_spec((HD, 2 * L)),
              _full_spec((1, HD))],
    out_specs=(_row_spec(HD), _row_spec(L), _row_spec(L)),
    out_shape=(jax.ShapeDtypeStruct((NPAD, HD), jnp.float32),
               jax.ShapeDtypeStruct((NPAD, L), jnp.float32),
               jax.ShapeDtypeStruct((NPAD, L), jnp.float32)),
)


def _block_alr(al, ar):
    eye = jnp.eye(H, dtype=jnp.float32)
    bl = (eye[:, None, :] * al[:, :, None]).reshape(HD, H)
    br = (eye[:, None, :] * ar[:, :, None]).reshape(HD, H)
    z = jnp.zeros((HD, H), jnp.float32)
    return jnp.concatenate([bl, z, br, z], axis=1)


@jax.jit
def kernel(g, feats, W0, al0, ar0, b0, gm0, bt0, W1, al1, ar1, b1, gm1, bt1,
           Wp, bp):
    src = jnp.concatenate(
        [g[0], jnp.full((E_ALLOC - E,), N, jnp.int32)]).astype(jnp.int32)
    dst = jnp.concatenate(
        [g[1], jnp.full((E_ALLOC - E,), N, jnp.int32)]).astype(jnp.int32)

    xpad = jnp.zeros((NPAD, IN), jnp.float32).at[:N].set(feats)

    p = jnp.zeros((L, HD), jnp.float32)
    eye = jnp.eye(H, dtype=jnp.float32)
    p = p.at[:H].set(jnp.repeat(eye, D, axis=1))

    alr0 = _block_alr(al0, ar0)
    alr1 = _block_alr(al1, ar1)

    wp_pad = jnp.zeros((HD, HD), jnp.float32).at[:, :OUT].set(Wp)
    bp_pad = jnp.zeros((HD,), jnp.float32).at[:OUT].set(bp)
    ws = jnp.stack([W1, wp_pad])
    alrs = jnp.stack([alr1, jnp.zeros((HD, 2 * L), jnp.float32)])
    bs = jnp.stack([b0[None, :], b1[None, :]])
    gms = jnp.stack([gm0[None, :], gm1[None, :]])
    bts = jnp.stack([bt0[None, :], bt1[None, :]])
    b2s = jnp.stack([jnp.zeros((1, HD), jnp.float32), bp_pad[None, :]])

    h0, la0, lb0 = _pre_call(xpad, W0, alr0)

    def body(carry, x):
        h, la, lb = carry
        w_i, alr_i, b_i, gm_i, bt_i, b2_i = x
        acc, s = _edge_call()(src, dst, h, la, lb)
        h2, la2, lb2 = _mid_call(acc, s, p, b_i, gm_i, bt_i, w_i, alr_i, b2_i)
        return (h2, la2, lb2), None

    (hf, _, _), _ = lax.scan(body, (h0, la0, lb0), (ws, alrs, bs, gms, bts, b2s))
    return hf[:N, :OUT]
